# baseline (speedup 1.0000x reference)
import functools

import jax
import jax.numpy as jnp
from jax import lax
from jax.experimental import pallas as pl
from jax.experimental.pallas import tpu as pltpu
from jax.experimental.pallas import tpu_sc as plsc

NC = 2
NS = 16
L = 16
NW = NC * NS
B = 16384
D = 64
BPW = B // NW
CHUNK = 128
NCHUNK = BPW // CHUNK
UNROLL = 8


def _als_body(user_hbm, item_hbm, uf_hbm, if_hbm, out_hbm,
              uidx_v, iidx_v, urows_v, irows_v, out_v, sem):
    wid = lax.axis_index("s") * NC + lax.axis_index("c")
    base = wid * BPW

    pltpu.sync_copy(user_hbm.at[pl.ds(base, BPW)], uidx_v)
    pltpu.sync_copy(item_hbm.at[pl.ds(base, BPW)], iidx_v)

    copies = []
    for j in range(NCHUNK):
        sl = pl.ds(j * CHUNK, CHUNK)
        copies.append(pltpu.async_copy(uf_hbm.at[uidx_v.at[sl]], urows_v.at[sl], sem))
        copies.append(pltpu.async_copy(if_hbm.at[iidx_v.at[sl]], irows_v.at[sl], sem))
    for c in copies:
        c.wait()

    iota = jax.lax.iota(jnp.int32, L)
    perms = [iota ^ sh for sh in (8, 4, 2, 1)]
    gdn = lax.GatherDimensionNumbers(offset_dims=(), collapsed_slice_dims=(0,),
                                     start_index_map=(0,))

    def lane_shuffle(x, perm):
        return lax.gather(x, perm[:, None], dimension_numbers=gdn,
                          slice_sizes=(1,),
                          mode=lax.GatherScatterMode.PROMISE_IN_BOUNDS)

    def body(g, carry):
        res = jnp.zeros((L,), jnp.float32)
        for l in range(L):
            p = g * L + l
            acc = urows_v[p, pl.ds(0, L)] * irows_v[p, pl.ds(0, L)]
            for k in range(1, D // L):
                acc = acc + urows_v[p, pl.ds(k * L, L)] * irows_v[p, pl.ds(k * L, L)]
            for perm in perms:
                acc = acc + lane_shuffle(acc, perm)
            res = jnp.where(iota == l, acc, res)
        out_v[pl.ds(g * L, L)] = res
        return carry

    lax.fori_loop(0, BPW // L, body, 0)

    pltpu.sync_copy(out_v, out_hbm.at[pl.ds(base, BPW)])


@functools.partial(
    pl.kernel,
    out_type=jax.ShapeDtypeStruct((B,), jnp.float32),
    mesh=plsc.VectorSubcoreMesh(core_axis_name="c", subcore_axis_name="s",
                                num_cores=NC, num_subcores=NS),
    scratch_types=[
        pltpu.VMEM((BPW,), jnp.int32),
        pltpu.VMEM((BPW,), jnp.int32),
        pltpu.VMEM((BPW, D), jnp.float32),
        pltpu.VMEM((BPW, D), jnp.float32),
        pltpu.VMEM((BPW,), jnp.float32),
        pltpu.SemaphoreType.DMA,
    ],
    compiler_params=pltpu.CompilerParams(use_tc_tiling_on_sc=False),
)
def _als_forward(user, item, uf, itf, out, *scratch):
    _als_body(user, item, uf, itf, out, *scratch)


def kernel(user, item, user_factors, item_factors):
    return _als_forward(user.astype(jnp.int32), item.astype(jnp.int32),
                        user_factors, item_factors)


# speedup vs baseline: 2.1837x; 2.1837x over previous
import functools

import jax
import jax.numpy as jnp
from jax import lax
from jax.experimental import pallas as pl
from jax.experimental.pallas import tpu as pltpu
from jax.experimental.pallas import tpu_sc as plsc

NC = 2
NS = 16
L = 16
NW = NC * NS
B = 16384
D = 64
BPW = B // NW
CH = 16
NCH = BPW // CH


def _als_body(user_hbm, item_hbm, uf_hbm, if_hbm, out_hbm,
              uidx_v, iidx_v, ubuf_v, ibuf_v, out_v, sem):
    wid = lax.axis_index("s") * NC + lax.axis_index("c")
    base = wid * BPW

    pltpu.sync_copy(user_hbm.at[pl.ds(base, BPW)], uidx_v)
    pltpu.sync_copy(item_hbm.at[pl.ds(base, BPW)], iidx_v)

    iota = jax.lax.iota(jnp.int32, L)
    perms = [iota ^ sh for sh in (8, 4, 2, 1)]
    gdn = lax.GatherDimensionNumbers(offset_dims=(), collapsed_slice_dims=(0,),
                                     start_index_map=(0,))

    def lane_shuffle(x, perm):
        return lax.gather(x, perm[:, None], dimension_numbers=gdn,
                          slice_sizes=(1,),
                          mode=lax.GatherScatterMode.PROMISE_IN_BOUNDS)

    def fire(c, slot):
        uvec = uidx_v[pl.ds(c * CH, CH)] >> 3
        ivec = iidx_v[pl.ds(c * CH, CH)] >> 3
        for j in range(CH):
            pltpu.async_copy(uf_hbm.at[uvec[j]], ubuf_v.at[slot, j], sem)
            pltpu.async_copy(if_hbm.at[ivec[j]], ibuf_v.at[slot, j], sem)

    fire(0, 0)

    def body(c, carry):
        slot = lax.rem(c, 2)

        @pl.when(c + 1 < NCH)
        def _():
            fire(c + 1, 1 - slot)

        pltpu.make_async_copy(uf_hbm.at[pl.ds(0, CH)],
                              ubuf_v.at[slot], sem).wait()
        pltpu.make_async_copy(if_hbm.at[pl.ds(0, CH)],
                              ibuf_v.at[slot], sem).wait()

        usub = uidx_v[pl.ds(c * CH, CH)] & 7
        isub = iidx_v[pl.ds(c * CH, CH)] & 7
        res = jnp.zeros((L,), jnp.float32)
        for j in range(CH):
            su = usub[j]
            si = isub[j]
            acc = (ubuf_v[slot, j, su, pl.ds(0, L)]
                   * ibuf_v[slot, j, si, pl.ds(0, L)])
            for k in range(1, D // L):
                acc = acc + (ubuf_v[slot, j, su, pl.ds(k * L, L)]
                             * ibuf_v[slot, j, si, pl.ds(k * L, L)])
            for perm in perms:
                acc = acc + lane_shuffle(acc, perm)
            res = jnp.where(iota == j, acc, res)
        out_v[pl.ds(c * CH, CH)] = res
        return carry

    lax.fori_loop(0, NCH, body, 0)

    pltpu.sync_copy(out_v, out_hbm.at[pl.ds(base, BPW)])


@functools.partial(
    pl.kernel,
    out_type=jax.ShapeDtypeStruct((B,), jnp.float32),
    mesh=plsc.VectorSubcoreMesh(core_axis_name="c", subcore_axis_name="s",
                                num_cores=NC, num_subcores=NS),
    scratch_types=[
        pltpu.VMEM((BPW,), jnp.int32),
        pltpu.VMEM((BPW,), jnp.int32),
        pltpu.VMEM((2, CH, 8, D), jnp.float32),
        pltpu.VMEM((2, CH, 8, D), jnp.float32),
        pltpu.VMEM((BPW,), jnp.float32),
        pltpu.SemaphoreType.DMA,
    ],
)
def _als_forward(user, item, uf, itf, out, *scratch):
    _als_body(user, item, uf, itf, out, *scratch)


def kernel(user, item, user_factors, item_factors):
    nu = user_factors.shape[0]
    ni = item_factors.shape[0]
    uf3 = user_factors.reshape(nu // 8, 8, D)
    if3 = item_factors.reshape(ni // 8, 8, D)
    return _als_forward(user.astype(jnp.int32), item.astype(jnp.int32),
                        uf3, if3)
